# baseline (speedup 1.0000x reference)
import functools

import jax
import jax.numpy as jnp
from jax import lax
from jax.experimental import pallas as pl
from jax.experimental.pallas import tpu as pltpu
from jax.experimental.pallas import tpu_sc as plsc

_K_PERCENT = 0.05
_NBINS = 2048
_NC = 2
_NS = 16
_NW = _NC * _NS
_NREP = 4


def _monotone(v):
    b = lax.bitcast_convert_type(v, jnp.int32)
    return b ^ ((b >> 31) & jnp.int32(0x7FFFFFFF))


def _zero_hists(hists):
    zeros = jnp.zeros((16,), jnp.float32)

    def zero(j, _):
        for h in hists:
            h[pl.ds(j * 16, 16)] = zeros
        return 0

    lax.fori_loop(0, _NBINS // 16, zero, 0)


def _merged(hists, o):
    acc = hists[0][pl.ds(o, 16)]
    for h in hists[1:]:
        acc = acc + h[pl.ds(o, 16)]
    return acc


def _sc_body(n, chunk, rows_per, k, x_hbm, peak_hbm,
             bufa, bufb, outbuf, sema, semb, *hists):
    hcnts = hists[:_NREP]
    hsums = hists[_NREP:]
    wid = lax.axis_index("s") * _NC + lax.axis_index("c")
    ones = jnp.full((16,), 1.0, jnp.float32)
    lane = lax.broadcasted_iota(jnp.int32, (16,), 0)
    kf = jnp.float32(k)
    z16 = jnp.zeros((16,), jnp.float32)
    group = 64
    nchunks = n // chunk
    bufs = (bufa, bufb)
    sems = (sema, semb)

    def stream(row, inner, init, unroll):
        acc = init
        h = pltpu.async_copy(x_hbm.at[pl.ds(row * n, chunk)], bufs[0], sems[0])
        for c in range(nchunks):
            h.wait()
            if c + 1 < nchunks:
                h = pltpu.async_copy(
                    x_hbm.at[pl.ds(row * n + (c + 1) * chunk, chunk)],
                    bufs[(c + 1) % 2], sems[(c + 1) % 2])
            acc = lax.fori_loop(0, chunk // group,
                                functools.partial(inner, bufs[c % 2]),
                                acc, unroll=unroll)
        return acc

    def do_row(r, _):
        row = wid * rows_per + r

        _zero_hists(hcnts)

        def step1(buf, j, _):
            base = j * group
            idxs = []
            for t in range(4):
                v = buf[pl.ds(base + t * 16, 16)]
                idxs.append((_monotone(v) >> 21) + 1024)
            for t in range(4):
                plsc.addupdate_scatter(hcnts[t], [idxs[t]], ones)
            return 0

        stream(row, step1, 0, unroll=16)

        def scan1(j, carry):
            cabove, sel_acc, cab_acc = carry
            o = (_NBINS // 16 - 1 - j) * 16
            c = _merged(hcnts, o)
            incl = plsc.cumsum(c)
            tot = jnp.sum(c)
            e = cabove + (tot - incl)
            m = (e < kf) & (e + c >= kf)
            sel_acc = sel_acc + jnp.where(m, (o + lane).astype(jnp.float32), 0.0)
            cab_acc = cab_acc + jnp.where(m, e, 0.0)
            return cabove + tot, sel_acc, cab_acc

        _, sel_acc, cab_acc = lax.fori_loop(
            0, _NBINS // 16, scan1, (jnp.float32(0.0), z16, z16))
        selv = jnp.full((16,), jnp.sum(sel_acc)).astype(jnp.int32)
        cab1 = jnp.sum(cab_acc)

        _zero_hists(hists)

        def step2(buf, j, accs_in):
            base = j * group
            vs, idxs, masks, gts = [], [], [], []
            for t in range(4):
                v = buf[pl.ds(base + t * 16, 16)]
                m = _monotone(v)
                b1 = (m >> 21) + 1024
                vs.append(v)
                masks.append(b1 == selv)
                gts.append(b1 > selv)
                idxs.append((m >> 10) & jnp.int32(0x7FF))
            accs_out = tuple(
                a + jnp.where(gts[t], vs[t], 0.0)
                for t, a in enumerate(accs_in))
            for t in range(4):
                plsc.addupdate_scatter(hcnts[t], [idxs[t]], ones,
                                       mask=masks[t])
                plsc.addupdate_scatter(hsums[t], [idxs[t]], vs[t],
                                       mask=masks[t])
            return accs_out

        accs = stream(row, step2, (z16, z16, z16, z16), unroll=8)
        s_above1 = jnp.sum(accs[0] + accs[1] + accs[2] + accs[3])
        r1 = kf - cab1

        def scan(j, carry):
            cc, sc, c2, s2, cst, sst = carry
            o = (_NBINS // 16 - 1 - j) * 16
            c = _merged(hcnts, o)
            s = _merged(hsums, o)
            incl_c = plsc.cumsum(c)
            incl_s = plsc.cumsum(s)
            tot_c = jnp.sum(c)
            tot_s = jnp.sum(s)
            e = cc + (tot_c - incl_c)
            es = sc + (tot_s - incl_s)
            m = (e < r1) & (e + c >= r1)
            c2 = c2 + jnp.where(m, e, 0.0)
            s2 = s2 + jnp.where(m, es, 0.0)
            cst = cst + jnp.where(m, c, 0.0)
            sst = sst + jnp.where(m, s, 0.0)
            return cc + tot_c, sc + tot_s, c2, s2, cst, sst

        _, _, c2, s2, cst, sst = lax.fori_loop(
            0, _NBINS // 16, scan,
            (jnp.float32(0.0), jnp.float32(0.0), z16, z16, z16, z16))
        c_ab2 = jnp.sum(c2)
        s_ab2 = jnp.sum(s2)
        cstar = jnp.sum(cst)
        sstar = jnp.sum(sst)
        rr = r1 - c_ab2
        out = jnp.where(lane == 0, jnp.full((16,), s_above1 + s_ab2), 0.0)
        out = jnp.where(lane == 1, jnp.full((16,), rr), out)
        out = jnp.where(lane == 2, jnp.full((16,), sstar), out)
        out = jnp.where(lane == 3, jnp.full((16,), cstar), out)
        outbuf[...] = out
        pltpu.sync_copy(outbuf, peak_hbm.at[pl.ds(row * 16, 16)])
        return 0

    lax.fori_loop(0, rows_per, do_row, 0)


def _loss_body(a_ref, r_ref, ss_ref, cs_ref, s_ref, t_ref, o_ref, *, nb, nc, k):
    mu = ss_ref[...] / jnp.maximum(cs_ref[...], 1.0)
    z = (a_ref[...] + r_ref[...] * mu) * jnp.float32(1.0 / k)
    s = s_ref[0, 0]
    sp = jnp.maximum(s, 0.0) + jnp.log(1.0 + jnp.exp(-jnp.abs(s)))
    z = z * sp
    m = jnp.max(z, axis=1, keepdims=True)
    lse = m + jnp.log(jnp.sum(jnp.exp(z - m), axis=1, keepdims=True))
    lp = z - lse
    cols = lax.broadcasted_iota(jnp.int32, (nb, nc), 1)
    sel = jnp.sum(jnp.where(cols == t_ref[...], lp, 0.0)) / nb
    o_ref[...] = jnp.full((8, 128), -sel, dtype=jnp.float32)


def kernel(inputs, scale, targets_class):
    B, C, H, W = inputs.shape
    n = H * W
    k = max(1, int(n * _K_PERCENT))
    rows = B * C
    assert rows % _NW == 0
    rows_per = rows // _NW
    chunk = 36864
    assert n % chunk == 0
    x1d = inputs.reshape(-1)

    mesh = plsc.VectorSubcoreMesh(core_axis_name="c", subcore_axis_name="s")
    vec_ty = jax.ShapeDtypeStruct((rows * 16,), jnp.float32)

    peaks = pl.kernel(
        functools.partial(_sc_body, n, chunk, rows_per, k),
        mesh=mesh,
        compiler_params=pltpu.CompilerParams(needs_layout_passes=False),
        out_type=vec_ty,
        scratch_types=(
            [
                pltpu.VMEM((chunk,), jnp.float32),
                pltpu.VMEM((chunk,), jnp.float32),
                pltpu.VMEM((16,), jnp.float32),
                pltpu.SemaphoreType.DMA,
                pltpu.SemaphoreType.DMA,
            ]
            + [pltpu.VMEM((_NBINS,), jnp.float32) for _ in range(2 * _NREP)]
        ),
    )(x1d)

    pk = peaks.reshape(rows, 16)
    parts = [pk[:, i].reshape(B, C) for i in range(4)]
    scale2d = scale.reshape(1, 1).astype(jnp.float32)
    tgt = targets_class.astype(jnp.int32).reshape(B, 1)

    loss = pl.pallas_call(
        functools.partial(_loss_body, nb=B, nc=C, k=k),
        in_specs=[
            pl.BlockSpec((B, C), lambda: (0, 0)),
            pl.BlockSpec((B, C), lambda: (0, 0)),
            pl.BlockSpec((B, C), lambda: (0, 0)),
            pl.BlockSpec((B, C), lambda: (0, 0)),
            pl.BlockSpec((1, 1), lambda: (0, 0)),
            pl.BlockSpec((B, 1), lambda: (0, 0)),
        ],
        out_specs=pl.BlockSpec((8, 128), lambda: (0, 0)),
        out_shape=jax.ShapeDtypeStruct((8, 128), jnp.float32),
    )(*parts, scale2d, tgt)

    return loss[0, 0]


# speedup vs baseline: 1.1335x; 1.1335x over previous
import functools

import jax
import jax.numpy as jnp
from jax import lax
from jax.experimental import pallas as pl
from jax.experimental.pallas import tpu as pltpu
from jax.experimental.pallas import tpu_sc as plsc

_K_PERCENT = 0.05
_NBINS = 2048
_NC = 2
_NS = 16
_NW = _NC * _NS
_NREP = 4


def _monotone(v):
    b = lax.bitcast_convert_type(v, jnp.int32)
    return b ^ ((b >> 31) & jnp.int32(0x7FFFFFFF))


def _zero_hists(hists):
    zeros = jnp.zeros((16,), jnp.float32)

    def zero(j, _):
        for h in hists:
            h[pl.ds(j * 16, 16)] = zeros
        return 0

    lax.fori_loop(0, _NBINS // 16, zero, 0)


def _merged(hists, o):
    acc = hists[0][pl.ds(o, 16)]
    for h in hists[1:]:
        acc = acc + h[pl.ds(o, 16)]
    return acc


def _sc_body(n, chunk, rows_per, k, x_hbm, peak_hbm,
             bufa, bufb, outbuf, sema, semb, *hists):
    hcnts = hists[:_NREP]
    hsums = hists[_NREP:]
    wid = lax.axis_index("s") * _NC + lax.axis_index("c")
    ones = jnp.full((16,), 1.0, jnp.float32)
    lane = lax.broadcasted_iota(jnp.int32, (16,), 0)
    kf = jnp.float32(k)
    z16 = jnp.zeros((16,), jnp.float32)
    group = 64
    nchunks = n // chunk
    bufs = (bufa, bufb)
    sems = (sema, semb)

    def stream(row, inner, init, unroll, grp):
        acc = init
        h = pltpu.async_copy(x_hbm.at[pl.ds(row * n, chunk)], bufs[0], sems[0])
        for c in range(nchunks):
            h.wait()
            if c + 1 < nchunks:
                h = pltpu.async_copy(
                    x_hbm.at[pl.ds(row * n + (c + 1) * chunk, chunk)],
                    bufs[(c + 1) % 2], sems[(c + 1) % 2])
            acc = lax.fori_loop(0, chunk // grp,
                                functools.partial(inner, bufs[c % 2]),
                                acc, unroll=unroll)
        return acc

    def do_row(r, _):
        row = wid * rows_per + r

        _zero_hists(hists)

        def step1(buf, j, _):
            base = j * 128
            idxs = []
            for t in range(8):
                v = buf[pl.ds(base + t * 16, 16)]
                idxs.append((_monotone(v) >> 21) + 1024)
            for t in range(8):
                plsc.addupdate_scatter(hists[t], [idxs[t]], ones)
            return 0

        stream(row, step1, 0, unroll=8, grp=128)

        def scan1(j, carry):
            cabove, sel_acc, cab_acc = carry
            o = (_NBINS // 16 - 1 - j) * 16
            c = _merged(hists, o)
            incl = plsc.cumsum(c)
            tot = jnp.sum(c)
            e = cabove + (tot - incl)
            m = (e < kf) & (e + c >= kf)
            sel_acc = sel_acc + jnp.where(m, (o + lane).astype(jnp.float32), 0.0)
            cab_acc = cab_acc + jnp.where(m, e, 0.0)
            return cabove + tot, sel_acc, cab_acc

        _, sel_acc, cab_acc = lax.fori_loop(
            0, _NBINS // 16, scan1, (jnp.float32(0.0), z16, z16))
        selv = jnp.full((16,), jnp.sum(sel_acc)).astype(jnp.int32)
        cab1 = jnp.sum(cab_acc)

        _zero_hists(hists)

        def step2(buf, j, accs_in):
            base = j * group
            vs, idxs, masks, gts = [], [], [], []
            for t in range(4):
                v = buf[pl.ds(base + t * 16, 16)]
                m = _monotone(v)
                b1 = (m >> 21) + 1024
                vs.append(v)
                masks.append(b1 == selv)
                gts.append(b1 > selv)
                idxs.append((m >> 10) & jnp.int32(0x7FF))
            accs_out = tuple(
                a + jnp.where(gts[t], vs[t], 0.0)
                for t, a in enumerate(accs_in))
            for t in range(4):
                plsc.addupdate_scatter(hcnts[t], [idxs[t]], ones,
                                       mask=masks[t])
                plsc.addupdate_scatter(hsums[t], [idxs[t]], vs[t],
                                       mask=masks[t])
            return accs_out

        accs = stream(row, step2, (z16, z16, z16, z16), unroll=8, grp=group)
        s_above1 = jnp.sum(accs[0] + accs[1] + accs[2] + accs[3])
        r1 = kf - cab1

        def scan(j, carry):
            cc, sc, c2, s2, cst, sst = carry
            o = (_NBINS // 16 - 1 - j) * 16
            c = _merged(hcnts, o)
            s = _merged(hsums, o)
            incl_c = plsc.cumsum(c)
            incl_s = plsc.cumsum(s)
            tot_c = jnp.sum(c)
            tot_s = jnp.sum(s)
            e = cc + (tot_c - incl_c)
            es = sc + (tot_s - incl_s)
            m = (e < r1) & (e + c >= r1)
            c2 = c2 + jnp.where(m, e, 0.0)
            s2 = s2 + jnp.where(m, es, 0.0)
            cst = cst + jnp.where(m, c, 0.0)
            sst = sst + jnp.where(m, s, 0.0)
            return cc + tot_c, sc + tot_s, c2, s2, cst, sst

        _, _, c2, s2, cst, sst = lax.fori_loop(
            0, _NBINS // 16, scan,
            (jnp.float32(0.0), jnp.float32(0.0), z16, z16, z16, z16))
        c_ab2 = jnp.sum(c2)
        s_ab2 = jnp.sum(s2)
        cstar = jnp.sum(cst)
        sstar = jnp.sum(sst)
        rr = r1 - c_ab2
        out = jnp.where(lane == 0, jnp.full((16,), s_above1 + s_ab2), 0.0)
        out = jnp.where(lane == 1, jnp.full((16,), rr), out)
        out = jnp.where(lane == 2, jnp.full((16,), sstar), out)
        out = jnp.where(lane == 3, jnp.full((16,), cstar), out)
        outbuf[...] = out
        pltpu.sync_copy(outbuf, peak_hbm.at[pl.ds(row * 16, 16)])
        return 0

    lax.fori_loop(0, rows_per, do_row, 0)


def _loss_body(a_ref, r_ref, ss_ref, cs_ref, s_ref, t_ref, o_ref, *, nb, nc, k):
    mu = ss_ref[...] / jnp.maximum(cs_ref[...], 1.0)
    z = (a_ref[...] + r_ref[...] * mu) * jnp.float32(1.0 / k)
    s = s_ref[0, 0]
    sp = jnp.maximum(s, 0.0) + jnp.log(1.0 + jnp.exp(-jnp.abs(s)))
    z = z * sp
    m = jnp.max(z, axis=1, keepdims=True)
    lse = m + jnp.log(jnp.sum(jnp.exp(z - m), axis=1, keepdims=True))
    lp = z - lse
    cols = lax.broadcasted_iota(jnp.int32, (nb, nc), 1)
    sel = jnp.sum(jnp.where(cols == t_ref[...], lp, 0.0)) / nb
    o_ref[...] = jnp.full((8, 128), -sel, dtype=jnp.float32)


def kernel(inputs, scale, targets_class):
    B, C, H, W = inputs.shape
    n = H * W
    k = max(1, int(n * _K_PERCENT))
    rows = B * C
    assert rows % _NW == 0
    rows_per = rows // _NW
    chunk = 36864
    assert n % chunk == 0
    x1d = inputs.reshape(-1)

    mesh = plsc.VectorSubcoreMesh(core_axis_name="c", subcore_axis_name="s")
    vec_ty = jax.ShapeDtypeStruct((rows * 16,), jnp.float32)

    peaks = pl.kernel(
        functools.partial(_sc_body, n, chunk, rows_per, k),
        mesh=mesh,
        compiler_params=pltpu.CompilerParams(needs_layout_passes=False),
        out_type=vec_ty,
        scratch_types=(
            [
                pltpu.VMEM((chunk,), jnp.float32),
                pltpu.VMEM((chunk,), jnp.float32),
                pltpu.VMEM((16,), jnp.float32),
                pltpu.SemaphoreType.DMA,
                pltpu.SemaphoreType.DMA,
            ]
            + [pltpu.VMEM((_NBINS,), jnp.float32) for _ in range(2 * _NREP)]
        ),
    )(x1d)

    pk = peaks.reshape(rows, 16)
    parts = [pk[:, i].reshape(B, C) for i in range(4)]
    scale2d = scale.reshape(1, 1).astype(jnp.float32)
    tgt = targets_class.astype(jnp.int32).reshape(B, 1)

    loss = pl.pallas_call(
        functools.partial(_loss_body, nb=B, nc=C, k=k),
        in_specs=[
            pl.BlockSpec((B, C), lambda: (0, 0)),
            pl.BlockSpec((B, C), lambda: (0, 0)),
            pl.BlockSpec((B, C), lambda: (0, 0)),
            pl.BlockSpec((B, C), lambda: (0, 0)),
            pl.BlockSpec((1, 1), lambda: (0, 0)),
            pl.BlockSpec((B, 1), lambda: (0, 0)),
        ],
        out_specs=pl.BlockSpec((8, 128), lambda: (0, 0)),
        out_shape=jax.ShapeDtypeStruct((8, 128), jnp.float32),
    )(*parts, scale2d, tgt)

    return loss[0, 0]


# speedup vs baseline: 1.2342x; 1.0889x over previous
import functools

import jax
import jax.numpy as jnp
from jax import lax
from jax.experimental import pallas as pl
from jax.experimental.pallas import tpu as pltpu
from jax.experimental.pallas import tpu_sc as plsc

_K_PERCENT = 0.05
_NBINS = 2048
_NC = 2
_NS = 16
_NW = _NC * _NS
_NREP = 4


def _monotone(v):
    b = lax.bitcast_convert_type(v, jnp.int32)
    return b ^ ((b >> 31) & jnp.int32(0x7FFFFFFF))


def _zero_hists(hists):
    zeros = jnp.zeros((16,), jnp.float32)

    def zero(j, _):
        for h in hists:
            h[pl.ds(j * 16, 16)] = zeros
        return 0

    lax.fori_loop(0, _NBINS // 16, zero, 0)


def _merged(hists, o):
    acc = hists[0][pl.ds(o, 16)]
    for h in hists[1:]:
        acc = acc + h[pl.ds(o, 16)]
    return acc


def _sc_body(n, chunk, rows_per, k, x_hbm, peak_hbm,
             bufa, bufb, outbuf, sema, semb, *hists):
    hcnts = hists[:_NREP]
    hsums = hists[_NREP:]
    wid = lax.axis_index("s") * _NC + lax.axis_index("c")
    ones = jnp.full((16,), 1.0, jnp.float32)
    lane = lax.broadcasted_iota(jnp.int32, (16,), 0)
    kf = jnp.float32(k)
    z16 = jnp.zeros((16,), jnp.float32)
    group = 64
    nchunks = n // chunk
    bufs = (bufa, bufb)
    sems = (sema, semb)

    def stream(row, inner, init, unroll, grp):
        acc = init
        h = pltpu.async_copy(x_hbm.at[pl.ds(row * n, chunk)], bufs[0], sems[0])
        for c in range(nchunks):
            h.wait()
            if c + 1 < nchunks:
                h = pltpu.async_copy(
                    x_hbm.at[pl.ds(row * n + (c + 1) * chunk, chunk)],
                    bufs[(c + 1) % 2], sems[(c + 1) % 2])
            acc = lax.fori_loop(0, chunk // grp,
                                functools.partial(inner, bufs[c % 2]),
                                acc, unroll=unroll)
        return acc

    def do_row(r, _):
        row = wid * rows_per + r

        _zero_hists(hists)

        def step1(buf, j, _):
            base = j * 128
            idxs = []
            for t in range(8):
                v = buf[pl.ds(base + t * 16, 16)]
                idxs.append((_monotone(v) >> 21) + 1024)
            for t in range(8):
                plsc.addupdate_scatter(hists[t], [idxs[t]], ones)
            return 0

        stream(row, step1, 0, unroll=8, grp=128)

        def scan1(j, carry):
            cabove, sel_acc, cab_acc = carry
            o = (_NBINS // 16 - 1 - j) * 16
            c = _merged(hists, o)
            incl = plsc.cumsum(c)
            tot = jnp.sum(c)
            e = cabove + (tot - incl)
            m = (e < kf) & (e + c >= kf)
            sel_acc = sel_acc + jnp.where(m, (o + lane).astype(jnp.float32), 0.0)
            cab_acc = cab_acc + jnp.where(m, e, 0.0)
            return cabove + tot, sel_acc, cab_acc

        _, sel_acc, cab_acc = lax.fori_loop(
            0, _NBINS // 16, scan1, (jnp.float32(0.0), z16, z16))
        selv = jnp.full((16,), jnp.sum(sel_acc)).astype(jnp.int32)
        cab1 = jnp.sum(cab_acc)

        _zero_hists(hists)

        def step2(buf, j, accs_in):
            base = j * 128
            vs, idxs, masks, gts = [], [], [], []
            for t in range(8):
                v = buf[pl.ds(base + t * 16, 16)]
                m = _monotone(v)
                b1 = (m >> 21) + 1024
                vs.append(v)
                masks.append(b1 == selv)
                gts.append(b1 > selv)
                idxs.append((m >> 10) & jnp.int32(0x7FF))
            accs_out = tuple(
                a + jnp.where(gts[t], vs[t], 0.0)
                for t, a in enumerate(accs_in))
            for t in range(8):
                plsc.addupdate_scatter(hcnts[t & 3], [idxs[t]], ones,
                                       mask=masks[t])
                plsc.addupdate_scatter(hsums[t & 3], [idxs[t]], vs[t],
                                       mask=masks[t])
            return accs_out

        accs = stream(row, step2, (z16,) * 8, unroll=8, grp=128)
        s_above1 = jnp.sum(sum(accs[1:], accs[0]))
        r1 = kf - cab1

        def scan(j, carry):
            cc, sc, c2, s2, cst, sst = carry
            o = (_NBINS // 16 - 1 - j) * 16
            c = _merged(hcnts, o)
            s = _merged(hsums, o)
            incl_c = plsc.cumsum(c)
            incl_s = plsc.cumsum(s)
            tot_c = jnp.sum(c)
            tot_s = jnp.sum(s)
            e = cc + (tot_c - incl_c)
            es = sc + (tot_s - incl_s)
            m = (e < r1) & (e + c >= r1)
            c2 = c2 + jnp.where(m, e, 0.0)
            s2 = s2 + jnp.where(m, es, 0.0)
            cst = cst + jnp.where(m, c, 0.0)
            sst = sst + jnp.where(m, s, 0.0)
            return cc + tot_c, sc + tot_s, c2, s2, cst, sst

        _, _, c2, s2, cst, sst = lax.fori_loop(
            0, _NBINS // 16, scan,
            (jnp.float32(0.0), jnp.float32(0.0), z16, z16, z16, z16))
        c_ab2 = jnp.sum(c2)
        s_ab2 = jnp.sum(s2)
        cstar = jnp.sum(cst)
        sstar = jnp.sum(sst)
        rr = r1 - c_ab2
        out = jnp.where(lane == 0, jnp.full((16,), s_above1 + s_ab2), 0.0)
        out = jnp.where(lane == 1, jnp.full((16,), rr), out)
        out = jnp.where(lane == 2, jnp.full((16,), sstar), out)
        out = jnp.where(lane == 3, jnp.full((16,), cstar), out)
        outbuf[...] = out
        pltpu.sync_copy(outbuf, peak_hbm.at[pl.ds(row * 16, 16)])
        return 0

    lax.fori_loop(0, rows_per, do_row, 0)


def _loss_body(a_ref, r_ref, ss_ref, cs_ref, s_ref, t_ref, o_ref, *, nb, nc, k):
    mu = ss_ref[...] / jnp.maximum(cs_ref[...], 1.0)
    z = (a_ref[...] + r_ref[...] * mu) * jnp.float32(1.0 / k)
    s = s_ref[0, 0]
    sp = jnp.maximum(s, 0.0) + jnp.log(1.0 + jnp.exp(-jnp.abs(s)))
    z = z * sp
    m = jnp.max(z, axis=1, keepdims=True)
    lse = m + jnp.log(jnp.sum(jnp.exp(z - m), axis=1, keepdims=True))
    lp = z - lse
    cols = lax.broadcasted_iota(jnp.int32, (nb, nc), 1)
    sel = jnp.sum(jnp.where(cols == t_ref[...], lp, 0.0)) / nb
    o_ref[...] = jnp.full((8, 128), -sel, dtype=jnp.float32)


def kernel(inputs, scale, targets_class):
    B, C, H, W = inputs.shape
    n = H * W
    k = max(1, int(n * _K_PERCENT))
    rows = B * C
    assert rows % _NW == 0
    rows_per = rows // _NW
    chunk = 36864
    assert n % chunk == 0
    x1d = inputs.reshape(-1)

    mesh = plsc.VectorSubcoreMesh(core_axis_name="c", subcore_axis_name="s")
    vec_ty = jax.ShapeDtypeStruct((rows * 16,), jnp.float32)

    peaks = pl.kernel(
        functools.partial(_sc_body, n, chunk, rows_per, k),
        mesh=mesh,
        compiler_params=pltpu.CompilerParams(needs_layout_passes=False),
        out_type=vec_ty,
        scratch_types=(
            [
                pltpu.VMEM((chunk,), jnp.float32),
                pltpu.VMEM((chunk,), jnp.float32),
                pltpu.VMEM((16,), jnp.float32),
                pltpu.SemaphoreType.DMA,
                pltpu.SemaphoreType.DMA,
            ]
            + [pltpu.VMEM((_NBINS,), jnp.float32) for _ in range(2 * _NREP)]
        ),
    )(x1d)

    pk = peaks.reshape(rows, 16)
    parts = [pk[:, i].reshape(B, C) for i in range(4)]
    scale2d = scale.reshape(1, 1).astype(jnp.float32)
    tgt = targets_class.astype(jnp.int32).reshape(B, 1)

    loss = pl.pallas_call(
        functools.partial(_loss_body, nb=B, nc=C, k=k),
        in_specs=[
            pl.BlockSpec((B, C), lambda: (0, 0)),
            pl.BlockSpec((B, C), lambda: (0, 0)),
            pl.BlockSpec((B, C), lambda: (0, 0)),
            pl.BlockSpec((B, C), lambda: (0, 0)),
            pl.BlockSpec((1, 1), lambda: (0, 0)),
            pl.BlockSpec((B, 1), lambda: (0, 0)),
        ],
        out_specs=pl.BlockSpec((8, 128), lambda: (0, 0)),
        out_shape=jax.ShapeDtypeStruct((8, 128), jnp.float32),
    )(*parts, scale2d, tgt)

    return loss[0, 0]
